# baseline (speedup 1.0000x reference)
import functools

import jax
import jax.numpy as jnp
from jax import lax
from jax.experimental import pallas as pl
from jax.experimental.pallas import tpu as pltpu
from jax.experimental.pallas import tpu_sc as plsc

N_NODES = 10000
N_EDGES = 320000
D = 128
H = D // 2

NC = 2
NS = 16

CHUNK = 80
CHUNKS_PER_TILE = N_EDGES // (NS * CHUNK)

NP = 10240
ROWS_PER_TILE = NP // NS

BM = 400


def _mm_body(x_ref, w1_ref, w2_ref, b_ref, t_ref):
    xb = x_ref[...]
    m1 = jnp.dot(xb, w1_ref[...], preferred_element_type=jnp.float32) + b_ref[...]
    m2 = jnp.dot(xb, w2_ref[...], preferred_element_type=jnp.float32)
    t_ref[0] = m1[:, :H]
    t_ref[1] = m1[:, H:]
    t_ref[2] = m2[:, :H]
    t_ref[3] = m2[:, H:]


def _precompute_table(x, w1, w2, b2d):
    return pl.pallas_call(
        _mm_body,
        grid=(N_NODES // BM,),
        in_specs=[
            pl.BlockSpec((BM, D), lambda i: (i, 0)),
            pl.BlockSpec((D, D), lambda i: (0, 0)),
            pl.BlockSpec((D, D), lambda i: (0, 0)),
            pl.BlockSpec((1, D), lambda i: (0, 0)),
        ],
        out_specs=pl.BlockSpec((4, BM, H), lambda i: (0, i, 0)),
        out_shape=jax.ShapeDtypeStruct((4, N_NODES, H), jnp.float32),
    )(x, w1, w2, b2d)


def _combine_body(p_ref, o_ref):
    o_ref[:, :H] = p_ref[0]
    o_ref[:, H:] = p_ref[1]


def _combine(partials):
    return pl.pallas_call(
        _combine_body,
        grid=(N_NODES // BM,),
        in_specs=[pl.BlockSpec((NC, BM, H), lambda i: (0, i, 0))],
        out_specs=pl.BlockSpec((BM, D), lambda i: (i, 0)),
        out_shape=jax.ShapeDtypeStruct((N_NODES, D), jnp.float32),
    )(partials)


@functools.partial(
    pl.kernel,
    out_type=jax.ShapeDtypeStruct((NC, NP, H), jnp.float32),
    mesh=plsc.VectorSubcoreMesh(core_axis_name="c", subcore_axis_name="s"),
    scratch_types=[
        pltpu.VMEM((CHUNKS_PER_TILE, CHUNK), jnp.int32),
        pltpu.VMEM((CHUNKS_PER_TILE, CHUNK), jnp.int32),
        pltpu.VMEM((3, CHUNK), jnp.int32),
        pltpu.VMEM((3, CHUNK, H), jnp.float32),
        pltpu.VMEM((3, CHUNK, H), jnp.float32),
        pltpu.VMEM_SHARED((NP, H), jnp.float32),
        pltpu.SemaphoreType.DMA((3,)),
        pltpu.SemaphoreType.DMA((3,)),
        pltpu.SemaphoreType.DMA((3,)),
    ],
    compiler_params=pltpu.CompilerParams(use_tc_tiling_on_sc=False),
)
def _sc_edges(t_hbm, srcg_hbm, dstg_hbm, out_hbm,
              sidx, didx, kidx, ra, rb, accum, sem_a, sem_b, sem_s):
    c = lax.axis_index("c")
    s = lax.axis_index("s")
    boff = (2 + c) * N_NODES

    zero = jnp.zeros((16,), jnp.float32)

    @pl.loop(0, CHUNK)
    def _zero_rows(e):
        for j in range(H // 16):
            ra[0, e, pl.ds(j * 16, 16)] = zero

    row0 = s * ROWS_PER_TILE

    @pl.loop(0, ROWS_PER_TILE // CHUNK)
    def _zero_accum(k):
        pltpu.sync_copy(ra.at[0], accum.at[pl.ds(row0 + k * CHUNK, CHUNK)])

    pltpu.sync_copy(srcg_hbm.at[c, s], sidx)
    pltpu.sync_copy(dstg_hbm.at[c, s], didx)

    plsc.subcore_barrier()

    def _issue(g, p):
        pltpu.async_copy(t_hbm.at[sidx.at[g]], ra.at[p], sem_a.at[p])
        pltpu.async_copy(t_hbm.at[didx.at[g]], rb.at[p], sem_b.at[p])

    def _wait_gather(g, p):
        pltpu.make_async_copy(t_hbm.at[sidx.at[g]], ra.at[p], sem_a.at[p]).wait()
        pltpu.make_async_copy(t_hbm.at[didx.at[g]], rb.at[p], sem_b.at[p]).wait()

    def _wait_scatter(p):
        pltpu.make_async_copy(ra.at[p], accum.at[kidx.at[p]], sem_s.at[p]).wait()

    @pl.loop(0, 2)
    def _prologue(g):
        _issue(g, g)

    @pl.loop(0, CHUNKS_PER_TILE)
    def _chunk(g):
        p = lax.rem(g, 3)
        _wait_gather(g, p)

        for j in range(CHUNK // 16):
            sl = pl.ds(j * 16, 16)
            kidx[p, sl] = didx[g, sl] - boff

        @pl.loop(0, CHUNK)
        def _row(e):
            for j in range(H // 16):
                sl = pl.ds(j * 16, 16)
                ra[p, e, sl] = jnp.maximum(ra[p, e, sl] + rb[p, e, sl], 0.0)

        pltpu.async_copy(ra.at[p], accum.at[kidx.at[p]], sem_s.at[p], add=True)

        @pl.when(g >= 1)
        def _drain_prev():
            _wait_scatter(lax.rem(g - 1, 3))

        @pl.when(g + 2 < CHUNKS_PER_TILE)
        def _prefetch():
            _issue(g + 2, lax.rem(g + 2, 3))

    _wait_scatter(lax.rem(CHUNKS_PER_TILE - 1, 3))

    plsc.subcore_barrier()

    pltpu.sync_copy(
        accum.at[pl.ds(row0, ROWS_PER_TILE)],
        out_hbm.at[c, pl.ds(row0, ROWS_PER_TILE)],
    )


def kernel(x, edge_index, W, b):
    w1 = W[:D]
    w2 = W[D:]
    b2d = b.reshape(1, D)
    table = _precompute_table(x, w1, w2, b2d).reshape(4 * N_NODES, H)
    src = edge_index[0].reshape(NS, CHUNKS_PER_TILE, CHUNK)
    dst = edge_index[1].reshape(NS, CHUNKS_PER_TILE, CHUNK)
    srcg = jnp.stack([src, src + N_NODES])
    dstg = jnp.stack([dst + 2 * N_NODES, dst + 3 * N_NODES])
    partials = _sc_edges(table, srcg, dstg)
    return _combine(partials)


# speedup vs baseline: 1.1429x; 1.1429x over previous
import functools

import jax
import jax.numpy as jnp
from jax import lax
from jax.experimental import pallas as pl
from jax.experimental.pallas import tpu as pltpu
from jax.experimental.pallas import tpu_sc as plsc

N_NODES = 10000
N_EDGES = 320000
D = 128
H = D // 2

NC = 2
NS = 16

CHUNK = 80
CHUNKS_PER_TILE = N_EDGES // (NS * CHUNK)

NP = 10240
ROWS_PER_TILE = NP // NS

BM = 400


def _mm_body(x_ref, w1_ref, w2_ref, b_ref, t_ref):
    xb = x_ref[...]
    m1 = jnp.dot(xb, w1_ref[...], preferred_element_type=jnp.float32) + b_ref[...]
    m2 = jnp.dot(xb, w2_ref[...], preferred_element_type=jnp.float32)
    t_ref[0] = m1[:, :H]
    t_ref[1] = m1[:, H:]
    t_ref[2] = m2[:, :H]
    t_ref[3] = m2[:, H:]


def _precompute_table(x, w1, w2, b2d):
    return pl.pallas_call(
        _mm_body,
        grid=(N_NODES // BM,),
        in_specs=[
            pl.BlockSpec((BM, D), lambda i: (i, 0)),
            pl.BlockSpec((D, D), lambda i: (0, 0)),
            pl.BlockSpec((D, D), lambda i: (0, 0)),
            pl.BlockSpec((1, D), lambda i: (0, 0)),
        ],
        out_specs=pl.BlockSpec((4, BM, H), lambda i: (0, i, 0)),
        out_shape=jax.ShapeDtypeStruct((4, N_NODES, H), jnp.float32),
    )(x, w1, w2, b2d)


def _combine_body(p_ref, o_ref):
    o_ref[:, :H] = p_ref[0]
    o_ref[:, H:] = p_ref[1]


def _combine(partials):
    return pl.pallas_call(
        _combine_body,
        grid=(N_NODES // BM,),
        in_specs=[pl.BlockSpec((NC, BM, H), lambda i: (0, i, 0))],
        out_specs=pl.BlockSpec((BM, D), lambda i: (i, 0)),
        out_shape=jax.ShapeDtypeStruct((N_NODES, D), jnp.float32),
    )(partials)


@functools.partial(
    pl.kernel,
    out_type=jax.ShapeDtypeStruct((NC, NP, H), jnp.float32),
    mesh=plsc.VectorSubcoreMesh(core_axis_name="c", subcore_axis_name="s"),
    scratch_types=[
        pltpu.VMEM((CHUNKS_PER_TILE, CHUNK), jnp.int32),
        pltpu.VMEM((CHUNKS_PER_TILE, CHUNK), jnp.int32),
        pltpu.VMEM((4, CHUNK), jnp.int32),
        pltpu.VMEM((4, CHUNK, H), jnp.float32),
        pltpu.VMEM((4, CHUNK, H), jnp.float32),
        pltpu.VMEM_SHARED((NP, H), jnp.float32),
        pltpu.SemaphoreType.DMA((4,)),
        pltpu.SemaphoreType.DMA((4,)),
        pltpu.SemaphoreType.DMA((4,)),
    ],
    compiler_params=pltpu.CompilerParams(use_tc_tiling_on_sc=False),
)
def _sc_edges(t_hbm, srcg_hbm, dstg_hbm, out_hbm,
              sidx, didx, kidx, ra, rb, accum, sem_a, sem_b, sem_s):
    c = lax.axis_index("c")
    s = lax.axis_index("s")
    boff = (2 + c) * N_NODES

    zero = jnp.zeros((16,), jnp.float32)

    @pl.loop(0, CHUNK)
    def _zero_rows(e):
        for j in range(H // 16):
            ra[0, e, pl.ds(j * 16, 16)] = zero

    row0 = s * ROWS_PER_TILE

    @pl.loop(0, ROWS_PER_TILE // CHUNK)
    def _zero_accum(k):
        pltpu.sync_copy(ra.at[0], accum.at[pl.ds(row0 + k * CHUNK, CHUNK)])

    pltpu.sync_copy(srcg_hbm.at[c, s], sidx)
    pltpu.sync_copy(dstg_hbm.at[c, s], didx)

    plsc.subcore_barrier()

    def _issue(g, k):
        pltpu.async_copy(t_hbm.at[sidx.at[g]], ra.at[k], sem_a.at[k])
        pltpu.async_copy(t_hbm.at[didx.at[g]], rb.at[k], sem_b.at[k])

    def _wait_gather(g, k):
        pltpu.make_async_copy(t_hbm.at[sidx.at[g]], ra.at[k], sem_a.at[k]).wait()
        pltpu.make_async_copy(t_hbm.at[didx.at[g]], rb.at[k], sem_b.at[k]).wait()

    def _wait_scatter(k):
        pltpu.make_async_copy(ra.at[k], accum.at[kidx.at[k]], sem_s.at[k]).wait()

    def _process(q, k):
        _wait_gather(q, k)
        for j in range(CHUNK // 16):
            sl = pl.ds(j * 16, 16)
            kidx[k, sl] = didx[q, sl] - boff

        @pl.loop(0, CHUNK, unroll=4)
        def _row(e):
            for j in range(H // 16):
                sl = pl.ds(j * 16, 16)
                ra[k, e, sl] = jnp.maximum(ra[k, e, sl] + rb[k, e, sl], 0.0)

        pltpu.async_copy(ra.at[k], accum.at[kidx.at[k]], sem_s.at[k], add=True)

    _issue(0, 0)
    _issue(1, 1)

    NG = CHUNKS_PER_TILE - 2

    @pl.loop(0, NG // 4)
    def _group(j):
        g = j * 4
        for k in range(4):
            q = g + k
            _process(q, k)
            kp = (k + 2) % 4

            @pl.when(q >= 2)
            def _drain():
                _wait_scatter(kp)

            _issue(q + 2, kp)

    _process(NG, 0)
    _process(NG + 1, 1)
    for k in (2, 3, 0, 1):
        _wait_scatter(k)

    plsc.subcore_barrier()

    pltpu.sync_copy(
        accum.at[pl.ds(row0, ROWS_PER_TILE)],
        out_hbm.at[c, pl.ds(row0, ROWS_PER_TILE)],
    )


def kernel(x, edge_index, W, b):
    w1 = W[:D]
    w2 = W[D:]
    b2d = b.reshape(1, D)
    table = _precompute_table(x, w1, w2, b2d).reshape(4 * N_NODES, H)
    src = edge_index[0].reshape(NS, CHUNKS_PER_TILE, CHUNK)
    dst = edge_index[1].reshape(NS, CHUNKS_PER_TILE, CHUNK)
    srcg = jnp.stack([src, src + N_NODES])
    dstg = jnp.stack([dst + 2 * N_NODES, dst + 3 * N_NODES])
    partials = _sc_edges(table, srcg, dstg)
    return _combine(partials)
